# Initial kernel scaffold; baseline (speedup 1.0000x reference)
import jax
import jax.numpy as jnp
from jax.experimental import pallas as pl


def kernel(inputs, table):
    raise NotImplementedError("write your pallas kernel here")


# speedup vs baseline: 1.7912x; 1.7912x over previous
import functools

import jax
import jax.numpy as jnp
from jax import lax
from jax.experimental import pallas as pl
from jax.experimental.pallas import tpu as pltpu
from jax.experimental.pallas import tpu_sc as plsc

D = 32


@functools.cache
def _make_gather(BATCH: int, HIST: int):
    info = plsc.get_sparse_core_info()
    NC, NS = info.num_cores, info.num_subcores
    NW = NC * NS
    bat_per_w = BATCH // NW
    GB = 16
    n_groups = bat_per_w // GB

    mesh = plsc.VectorSubcoreMesh(core_axis_name="c", subcore_axis_name="s")

    @functools.partial(
        pl.kernel,
        mesh=mesh,
        compiler_params=pltpu.CompilerParams(use_tc_tiling_on_sc=False),
        out_type=jax.ShapeDtypeStruct((BATCH, HIST, D), jnp.float32),
        scratch_types=[
            pltpu.VMEM((bat_per_w, HIST), jnp.int32),
            pltpu.VMEM((2, GB, HIST, D), jnp.float32),
            pltpu.SemaphoreType.DMA,
        ],
    )
    def k(table_hbm, idx_hbm, out_hbm, idx_v, stage, gsem):
        wid = lax.axis_index("s") * NC + lax.axis_index("c")
        bat_base = wid * bat_per_w
        pltpu.sync_copy(idx_hbm.at[pl.ds(bat_base, bat_per_w)], idx_v)

        def body(g, carry):
            buf = lax.rem(g, 2)
            handles = [
                pltpu.async_copy(
                    table_hbm.at[idx_v.at[g * GB + b]],
                    stage.at[buf, b],
                    gsem,
                )
                for b in range(GB)
            ]

            @pl.when(g > 0)
            def _():
                pltpu.sync_copy(
                    stage.at[1 - buf],
                    out_hbm.at[pl.ds(bat_base + (g - 1) * GB, GB)],
                )

            for h in handles:
                h.wait()
            return carry

        lax.fori_loop(0, n_groups, body, 0)
        pltpu.sync_copy(
            stage.at[(n_groups - 1) % 2],
            out_hbm.at[pl.ds(bat_base + (n_groups - 1) * GB, GB)],
        )

    return k


def kernel(inputs, table):
    BATCH, HIST = inputs.shape
    return _make_gather(BATCH, HIST)(table, inputs.astype(jnp.int32))
